# baseline (speedup 1.0000x reference)
import functools

import jax
import jax.numpy as jnp
from jax import lax
from jax.experimental import pallas as pl
from jax.experimental.pallas import tpu as pltpu
from jax.experimental.pallas import tpu_sc as plsc

_N = 100000
_D = 64
_OUT_ROWS = 3 * _N

_GROUPS = 1563
_G_ROWS = _GROUPS * 128

_CHUNK = 128
_CHUNKS = 49
_TILE_ROWS = _CHUNK * _CHUNKS
_TILE_STRIDE = 6256
_LAST_BASE = _G_ROWS - _TILE_ROWS

_SEAM = 781
_ALL_BLOCKS = 2344


def _sc_gather_body(ctab, idx, out, idx_v, row_a, row_b, sem_a, sem_b):
    info = plsc.get_sparse_core_info()
    nc = info.num_cores
    wid = lax.axis_index("s") * nc + lax.axis_index("c")
    base = jnp.minimum(wid * _TILE_STRIDE, _LAST_BASE)

    pltpu.sync_copy(idx.at[pl.ds(base, _TILE_ROWS)], idx_v)

    def step(j, _):
        off = j * _CHUNK
        pltpu.async_copy(ctab.at[idx_v.at[pl.ds(off, _CHUNK)]],
                         row_a, sem_a).wait()
        pltpu.sync_copy(row_a, out.at[pl.ds(base + off, _CHUNK)])
        return _

    lax.fori_loop(0, _CHUNKS, step, 0)


def _sc_gather(ctab, idx):
    mesh = plsc.VectorSubcoreMesh(core_axis_name="c", subcore_axis_name="s")
    fn = functools.partial(
        pl.kernel,
        mesh=mesh,
        compiler_params=pltpu.CompilerParams(use_tc_tiling_on_sc=False),
        out_type=jax.ShapeDtypeStruct((_G_ROWS, _D), jnp.float32),
        scratch_types=[
            pltpu.VMEM((_TILE_ROWS,), jnp.int32),
            pltpu.VMEM((_CHUNK, _D), jnp.float32),
            pltpu.VMEM((_CHUNK, _D), jnp.float32),
            pltpu.SemaphoreType.DMA,
            pltpu.SemaphoreType.DMA,
        ],
    )(_sc_gather_body)
    return fn(ctab, idx)


def _tc_body(ipt_ref, wt_ref, bt_ref, g2_ref, o_ref):
    i = pl.program_id(0)

    def mm():
        return jnp.dot(wt_ref[...], ipt_ref[...],
                       preferred_element_type=jnp.float32) + bt_ref[...]

    def tr():
        t = jnp.transpose(g2_ref[...])
        return jnp.concatenate([t[0:_D, :], t[_D:128, :]], axis=1)

    @pl.when(i < _SEAM)
    def _ip():
        o_ref[...] = mm()

    @pl.when(i == _SEAM)
    def _seam():
        lane = lax.broadcasted_iota(jnp.int32, (_D, 128), 1)
        o_ref[...] = jnp.where(lane < 32, mm(), tr())

    @pl.when(i > _SEAM)
    def _gather():
        o_ref[...] = tr()


def _tc_assemble(ipt, wt, bt, g2):
    return pl.pallas_call(
        _tc_body,
        grid=(_ALL_BLOCKS,),
        in_specs=[
            pl.BlockSpec((32, 128), lambda i: (0, jnp.minimum(i, _SEAM))),
            pl.BlockSpec((_D, 32), lambda i: (0, 0)),
            pl.BlockSpec((_D, 1), lambda i: (0, 0)),
            pl.BlockSpec((_D, 128), lambda i: (jnp.maximum(i - _SEAM, 0), 0)),
        ],
        out_specs=pl.BlockSpec((_D, 128), lambda i: (0, i)),
        out_shape=jax.ShapeDtypeStruct((_D, _OUT_ROWS), jnp.float32),
        compiler_params=pltpu.CompilerParams(
            dimension_semantics=("arbitrary",),
        ),
    )(ipt, wt, bt, g2)


def kernel(ip_feats, port_idx, tech_idx, W_ip, b_ip, port_table, tech_table):
    ctab = jnp.concatenate([port_table, tech_table], axis=0)
    zpad = jnp.zeros((32,), jnp.int32)
    src = jnp.concatenate([zpad, port_idx.astype(jnp.int32),
                           tech_idx.astype(jnp.int32) + 65536, zpad])
    idx_g = src.reshape(_GROUPS, 2, 64).transpose(0, 2, 1).reshape(-1)
    g = _sc_gather(ctab, idx_g)
    g2 = g.reshape(_G_ROWS // 2, 128)
    out_t = _tc_assemble(ip_feats.T, W_ip.T, b_ip.reshape(_D, 1), g2)
    return out_t.T


# speedup vs baseline: 4.1749x; 4.1749x over previous
import functools

import jax
import jax.numpy as jnp
from jax import lax
from jax.experimental import pallas as pl
from jax.experimental.pallas import tpu as pltpu
from jax.experimental.pallas import tpu_sc as plsc

_N = 100000
_D = 64
_OUT_ROWS = 3 * _N

_BN = 1024
_GROUPS = 196
_G_ROWS = _GROUPS * _BN

_CHUNK = 128
_CHUNKS = 49
_TILE_ROWS = _CHUNK * _CHUNKS

_SEAM = 97
_ALL_BLOCKS = 293
_SEAM_IP = _N - _SEAM * _BN


def _sc_gather_body(ctab, idx, out, idx_v, row_a, row_b, sem_a, sem_b):
    info = plsc.get_sparse_core_info()
    nc = info.num_cores
    wid = lax.axis_index("s") * nc + lax.axis_index("c")
    base = wid * _TILE_ROWS

    pltpu.sync_copy(idx.at[pl.ds(base, _TILE_ROWS)], idx_v)

    bufs = (row_a, row_b)
    sems = (sem_a, sem_b)

    def start(chunk, b):
        pltpu.async_copy(ctab.at[idx_v.at[pl.ds(chunk * _CHUNK, _CHUNK)]],
                         bufs[b], sems[b])

    def drain(b):
        pltpu.make_async_copy(ctab.at[pl.ds(0, _CHUNK)],
                              bufs[b], sems[b]).wait()

    def write(chunk, b):
        pltpu.sync_copy(bufs[b], out.at[pl.ds(base + chunk * _CHUNK, _CHUNK)])

    start(0, 0)
    start(1, 1)

    def step(j, carry):
        for b in range(2):
            drain(b)
            write(j + b, b)

            @pl.when(j + b + 2 < _CHUNKS)
            def _start_next(b=b, j=j):
                start(j + b + 2, b)
        return carry

    lax.fori_loop(0, _CHUNKS // 2, lambda j, c: step(2 * j, c), 0)
    drain(0)
    write(_CHUNKS - 1, 0)


def _sc_gather(ctab, idx):
    mesh = plsc.VectorSubcoreMesh(core_axis_name="c", subcore_axis_name="s")
    fn = functools.partial(
        pl.kernel,
        mesh=mesh,
        compiler_params=pltpu.CompilerParams(use_tc_tiling_on_sc=False),
        out_type=jax.ShapeDtypeStruct((_G_ROWS, _D), jnp.float32),
        scratch_types=[
            pltpu.VMEM((_TILE_ROWS,), jnp.int32),
            pltpu.VMEM((_CHUNK, _D), jnp.float32),
            pltpu.VMEM((_CHUNK, _D), jnp.float32),
            pltpu.SemaphoreType.DMA,
            pltpu.SemaphoreType.DMA,
        ],
    )(_sc_gather_body)
    return fn(ctab, idx)


def _mm_body(ipt_ref, wt_ref, bt_ref, o_ref):
    o_ref[...] = jnp.dot(wt_ref[...], ipt_ref[...],
                         preferred_element_type=jnp.float32) + bt_ref[...]


def _mm_kernel(ipt, wt, bt):
    return pl.pallas_call(
        _mm_body,
        grid=(_SEAM + 1,),
        in_specs=[
            pl.BlockSpec((32, _BN), lambda i: (0, i)),
            pl.BlockSpec((_D, 32), lambda i: (0, 0)),
            pl.BlockSpec((_D, 1), lambda i: (0, 0)),
        ],
        out_specs=pl.BlockSpec((_D, _BN), lambda i: (0, i)),
        out_shape=jax.ShapeDtypeStruct((_D, _OUT_ROWS), jnp.float32),
        compiler_params=pltpu.CompilerParams(
            dimension_semantics=("arbitrary",),
        ),
    )(ipt, wt, bt)


def _tr_body(z_ref, g2_ref, o_ref):
    j = pl.program_id(0)
    t = jnp.transpose(g2_ref[...])

    @pl.when(j == 0)
    def _seam():
        lane = lax.broadcasted_iota(jnp.int32, (_D, _BN), 1)
        tcat = jnp.concatenate([t[0:_D, :], t[_D:128, :]], axis=1)
        o_ref[...] = jnp.where(lane < _SEAM_IP, z_ref[...], tcat)

    @pl.when(j > 0)
    def _full():
        o_ref[:, 0:_BN // 2] = t[0:_D, :]
        o_ref[:, _BN // 2:_BN] = t[_D:128, :]


def _tr_kernel(z, g2):
    return pl.pallas_call(
        _tr_body,
        grid=(_ALL_BLOCKS - _SEAM,),
        in_specs=[
            pl.BlockSpec((_D, _BN), lambda j: (0, _SEAM)),
            pl.BlockSpec((_BN // 2, 128), lambda j: (j, 0)),
        ],
        out_specs=pl.BlockSpec((_D, _BN), lambda j: (0, j + _SEAM)),
        out_shape=jax.ShapeDtypeStruct((_D, _OUT_ROWS), jnp.float32),
        input_output_aliases={0: 0},
        compiler_params=pltpu.CompilerParams(
            dimension_semantics=("arbitrary",),
        ),
    )(z, g2)


def kernel(ip_feats, port_idx, tech_idx, W_ip, b_ip, port_table, tech_table):
    ctab = jnp.concatenate([port_table, tech_table], axis=0)
    head = jnp.zeros((_SEAM_IP,), jnp.int32)
    tail = jnp.zeros((_SEAM * _BN + _G_ROWS - 3 * _N,), jnp.int32)
    src = jnp.concatenate([head, port_idx.astype(jnp.int32),
                           tech_idx.astype(jnp.int32) + 65536, tail])
    idx_g = src.reshape(_GROUPS, 2, _BN // 2).transpose(0, 2, 1).reshape(-1)
    g = _sc_gather(ctab, idx_g)
    g2 = g.reshape(_G_ROWS // 2, 128)
    z = _mm_kernel(ip_feats.T, W_ip.T, b_ip.reshape(_D, 1))
    out_t = _tr_kernel(z, g2)
    return out_t.T


# speedup vs baseline: 4.9129x; 1.1768x over previous
import functools

import jax
import jax.numpy as jnp
from jax import lax
from jax.experimental import pallas as pl
from jax.experimental.pallas import tpu as pltpu
from jax.experimental.pallas import tpu_sc as plsc

_N = 100000
_D = 64
_OUT_ROWS = 3 * _N

_BN = 1024
_GROUPS = 196
_G_ROWS = _GROUPS * _BN

_CHUNK = 128
_CHUNKS = 49
_TILE_ROWS = _CHUNK * _CHUNKS
_SRC_SPAN = 8192

_SEAM = 97
_ALL_BLOCKS = 293
_SEAM_IP = _N - _SEAM * _BN


def _sc_gather_body(ctab, src, out, src_v, didx_v,
                    row_a, row_b, sem_a, sem_b, sem_w):
    info = plsc.get_sparse_core_info()
    nc = info.num_cores
    wid = lax.axis_index("s") * nc + lax.axis_index("c")
    base = wid * _TILE_ROWS
    origin = (base >> 10) << 10
    origin = pl.multiple_of(origin, 1024)

    pltpu.sync_copy(src.at[pl.ds(origin, _SRC_SPAN)], src_v)

    bufs = (row_a, row_b)
    sems = (sem_a, sem_b)
    lanes = lax.iota(jnp.int32, 16)

    def start(chunk, b):
        r0 = base + chunk * _CHUNK
        la = ((r0 >> 10) << 10) + 64 * ((r0 >> 7) & 7) - origin
        la = pl.multiple_of(la, 64)
        pltpu.async_copy(ctab.at[src_v.at[pl.ds(la, 64)]],
                         bufs[b].at[pl.ds(0, 64)], sems[b])
        pltpu.async_copy(ctab.at[src_v.at[pl.ds(la + 512, 64)]],
                         bufs[b].at[pl.ds(64, 64)], sems[b])

    def drain(b):
        pltpu.make_async_copy(ctab.at[pl.ds(0, _CHUNK)],
                              bufs[b], sems[b]).wait()

    def write(chunk, b):
        r0 = base + chunk * _CHUNK
        for v in range(_CHUNK // 16):
            pos = lanes + 16 * v
            didx_v[pl.ds(16 * v, 16)] = jnp.where(
                pos < 64, r0 + 2 * pos, r0 + 2 * (pos - 64) + 1)
        pltpu.async_copy(bufs[b], out.at[didx_v], sem_w).wait()

    start(0, 0)
    start(1, 1)

    def step(j, carry):
        for b in range(2):
            drain(b)
            write(j + b, b)

            @pl.when(j + b + 2 < _CHUNKS)
            def _start_next(b=b, j=j):
                start(j + b + 2, b)
        return carry

    lax.fori_loop(0, _CHUNKS // 2, lambda j, c: step(2 * j, c), 0)
    drain(0)
    write(_CHUNKS - 1, 0)


def _sc_gather(ctab, idx):
    mesh = plsc.VectorSubcoreMesh(core_axis_name="c", subcore_axis_name="s")
    fn = functools.partial(
        pl.kernel,
        mesh=mesh,
        compiler_params=pltpu.CompilerParams(use_tc_tiling_on_sc=False),
        out_type=jax.ShapeDtypeStruct((_G_ROWS, _D), jnp.float32),
        scratch_types=[
            pltpu.VMEM((_SRC_SPAN,), jnp.int32),
            pltpu.VMEM((_CHUNK,), jnp.int32),
            pltpu.VMEM((_CHUNK, _D), jnp.float32),
            pltpu.VMEM((_CHUNK, _D), jnp.float32),
            pltpu.SemaphoreType.DMA,
            pltpu.SemaphoreType.DMA,
            pltpu.SemaphoreType.DMA,
        ],
    )(_sc_gather_body)
    return fn(ctab, idx)


def _mm_body(ipt_ref, wt_ref, bt_ref, o_ref):
    o_ref[...] = jnp.dot(wt_ref[...], ipt_ref[...],
                         preferred_element_type=jnp.float32) + bt_ref[...]


def _mm_kernel(ipt, wt, bt):
    return pl.pallas_call(
        _mm_body,
        grid=(_SEAM + 1,),
        in_specs=[
            pl.BlockSpec((32, _BN), lambda i: (0, i)),
            pl.BlockSpec((_D, 32), lambda i: (0, 0)),
            pl.BlockSpec((_D, 1), lambda i: (0, 0)),
        ],
        out_specs=pl.BlockSpec((_D, _BN), lambda i: (0, i)),
        out_shape=jax.ShapeDtypeStruct((_D, _OUT_ROWS), jnp.float32),
        compiler_params=pltpu.CompilerParams(
            dimension_semantics=("arbitrary",),
        ),
    )(ipt, wt, bt)


def _tr_body(z_ref, g2_ref, o_ref):
    j = pl.program_id(0)
    t = jnp.transpose(g2_ref[...])

    @pl.when(j == 0)
    def _seam():
        lane = lax.broadcasted_iota(jnp.int32, (_D, _BN), 1)
        tcat = jnp.concatenate([t[0:_D, :], t[_D:128, :]], axis=1)
        o_ref[...] = jnp.where(lane < _SEAM_IP, z_ref[...], tcat)

    @pl.when(j > 0)
    def _full():
        o_ref[:, 0:_BN // 2] = t[0:_D, :]
        o_ref[:, _BN // 2:_BN] = t[_D:128, :]


def _tr_kernel(z, g2):
    return pl.pallas_call(
        _tr_body,
        grid=(_ALL_BLOCKS - _SEAM,),
        in_specs=[
            pl.BlockSpec((_D, _BN), lambda j: (0, _SEAM)),
            pl.BlockSpec((_BN // 2, 128), lambda j: (j, 0)),
        ],
        out_specs=pl.BlockSpec((_D, _BN), lambda j: (0, j + _SEAM)),
        out_shape=jax.ShapeDtypeStruct((_D, _OUT_ROWS), jnp.float32),
        input_output_aliases={0: 0},
        compiler_params=pltpu.CompilerParams(
            dimension_semantics=("arbitrary",),
        ),
    )(z, g2)


def kernel(ip_feats, port_idx, tech_idx, W_ip, b_ip, port_table, tech_table):
    ctab = jnp.concatenate([port_table, tech_table], axis=0)
    head = jnp.zeros((_SEAM_IP,), jnp.int32)
    tail = jnp.zeros((_SEAM * _BN + _G_ROWS - 3 * _N + 1024,), jnp.int32)
    src = jnp.concatenate([head, port_idx.astype(jnp.int32),
                           tech_idx.astype(jnp.int32) + 65536, tail])
    g = _sc_gather(ctab, src)
    g2 = g.reshape(_G_ROWS // 2, 128)
    z = _mm_kernel(ip_feats.T, W_ip.T, b_ip.reshape(_D, 1))
    out_t = _tr_kernel(z, g2)
    return out_t.T
